# baseline (device time: 131201 ns/iter reference)
import jax
import jax.numpy as jnp
from jax import lax
from jax.experimental import pallas as pl
from jax.experimental.pallas import tpu as pltpu

N_DEV = 4
SQ = 1024
SKV = 1024
D_MODEL = 1024
HB = 8
DH = 128
BLK = HB * DH
G = 4
GS = SKV // G
SCALE = 0.08838834764831843

_F32 = jnp.float32
_BF16 = jnp.bfloat16

_MESH = pl.DeviceIdType.MESH


def _body(x_ref, wq_ref, k_hbm, v_hbm, wo_ref, out_ref,
          wq_buf, wo_buf, k_buf, v_buf, ctx_hop, ctx_keep,
          send_sems, recv_sems, kv_sems):
    my = lax.axis_index("i")
    left = lax.rem(my + (N_DEV - 1), N_DEV)
    right = lax.rem(my + 1, N_DEV)

    barrier = pltpu.get_barrier_semaphore()
    for nbr in (left, right):
        pl.semaphore_signal(barrier, inc=1, device_id=(nbr,),
                            device_id_type=_MESH)
    pl.semaphore_wait(barrier, 2)

    def wq_send(h, src):
        d = pltpu.make_async_remote_copy(
            src_ref=src, dst_ref=wq_buf.at[h],
            send_sem=send_sems.at[0, h], recv_sem=recv_sems.at[0, h],
            device_id=(right,), device_id_type=_MESH)
        d.start()
        return d

    def wo_send(h, src):
        d = pltpu.make_async_remote_copy(
            src_ref=src, dst_ref=wo_buf.at[h],
            send_sem=send_sems.at[1, h], recv_sem=recv_sems.at[1, h],
            device_id=(left,), device_id_type=_MESH)
        d.start()
        return d

    def kv_start(slot, blk):
        ds = []
        for r in range(G):
            ds.append(pltpu.make_async_copy(
                k_hbm.at[:, r, :, pl.ds(blk * BLK, BLK)],
                k_buf.at[slot, r], kv_sems.at[0, slot]))
            ds.append(pltpu.make_async_copy(
                v_hbm.at[:, r, :, pl.ds(blk * BLK, BLK)],
                v_buf.at[slot, r], kv_sems.at[1, slot]))
        for d in ds:
            d.start()
        return ds

    def kv_wait(ds):
        for d in ds:
            d.wait()

    def attn(wq_src, slot, ctx_dst, c0=0, nh=HB):
        q_full = lax.dot_general(x_ref[...], wq_src,
                                 (((1,), (0,)), ((), ())),
                                 preferred_element_type=_F32)
        for hd in range(nh):
            c = c0 + hd * DH
            q3 = (q_full[:, hd * DH:(hd + 1) * DH]
                  .astype(_BF16).reshape(G, GS, DH))
            k3 = (k_buf[slot, :, :, :, c:c + DH]
                  .astype(_BF16).reshape(G, GS, DH))
            v3 = (v_buf[slot, :, :, :, c:c + DH]
                  .astype(_BF16).reshape(G, GS, DH))
            s = lax.dot_general(q3, k3, (((2,), (2,)), ((0,), (0,))),
                                preferred_element_type=_F32)
            p = jnp.exp(s)
            p = (p * (1.0 / jnp.sum(p, axis=2, keepdims=True))).astype(_BF16)
            ctx = lax.dot_general(p, v3, (((2,), (1,)), ((0,), (0,))),
                                  preferred_element_type=_F32)
            ctx_dst[:, c:c + DH] = ctx.astype(_BF16).reshape(SQ, DH)

    def outproj(ctx_src, wo_src, first=False, col0=0):
        y = lax.dot_general(ctx_src[...], wo_src,
                            (((1,), (0,)), ((), ())),
                            preferred_element_type=_F32)
        w = y.shape[1]
        if first:
            out_ref[:, col0:col0 + w] = y
        else:
            out_ref[:, col0:col0 + w] += y

    snd = [wq_send(0, wq_ref), wo_send(0, wo_ref)]
    kv0 = kv_start(0, my)
    kv1 = kv_start(1, lax.rem(my + 3, N_DEV))

    kv_wait(kv0)
    attn(wq_ref[...], 0, ctx_hop)
    outproj(ctx_hop, wo_ref[...], first=True)

    snd[0].wait_recv()
    snd[1].wait_recv()
    snd.append(wq_send(1, wq_buf.at[0]))
    snd.append(wo_send(1, wo_buf.at[0]))
    kv2 = kv_start(0, lax.rem(my + 2, N_DEV))
    kv_wait(kv1)
    attn(wq_buf[0], 1, ctx_keep)

    snd[2].wait_recv()
    snd[3].wait_recv()
    halves = []
    for hf in range(2):
        cs = pl.ds(hf * 512, 512)
        d = pltpu.make_async_remote_copy(
            src_ref=wq_buf.at[1, :, cs], dst_ref=wq_buf.at[2, :, cs],
            send_sem=send_sems.at[0, 2 + hf],
            recv_sem=recv_sems.at[0, 2 + hf],
            device_id=(right,), device_id_type=_MESH)
        d.start()
        halves.append(d)
        snd.append(d)
        d = pltpu.make_async_remote_copy(
            src_ref=wo_buf.at[1, :, cs], dst_ref=wo_buf.at[2, :, cs],
            send_sem=send_sems.at[1, 2 + hf],
            recv_sem=recv_sems.at[1, 2 + hf],
            device_id=(left,), device_id_type=_MESH)
        d.start()
        halves.append(d)
        snd.append(d)
    kv3 = kv_start(1, lax.rem(my + 1, N_DEV))
    kv_wait(kv2)
    attn(wq_buf[1], 0, ctx_hop)
    outproj(ctx_hop, wo_buf[1])

    kv_wait(kv3)
    halves[0].wait_recv()
    attn(wq_buf[2, :, 0:512], 1, ctx_hop, 0, HB // 2)
    halves[1].wait_recv()
    outproj(ctx_keep, wo_buf[2, :, 0:512], col0=0)
    halves[2].wait_recv()
    attn(wq_buf[2, :, 512:1024], 1, ctx_hop, 512, HB // 2)
    halves[3].wait_recv()
    outproj(ctx_keep, wo_buf[2, :, 512:1024], col0=512)
    outproj(ctx_hop, wo_buf[0])

    for d in snd:
        d.wait_send()


def _perm(a):
    n = a.shape[0]
    return a.reshape(G, G, 64, -1).transpose(1, 0, 2, 3).reshape(n, -1)


def kernel(x, Wq, K_ext, V_ext, Wo):
    x2 = _perm(x.reshape(SQ, D_MODEL)).astype(_BF16)
    wq = (Wq * SCALE).astype(_BF16)
    wo = Wo.astype(_BF16)
    k2 = K_ext.reshape(G, G, 64, N_DEV * BLK)
    v2 = V_ext.reshape(G, G, 64, N_DEV * BLK)
    out = pl.pallas_call(
        _body,
        out_shape=jax.ShapeDtypeStruct((SQ, D_MODEL), _F32),
        in_specs=[
            pl.BlockSpec(memory_space=pltpu.VMEM),
            pl.BlockSpec(memory_space=pltpu.VMEM),
            pl.BlockSpec(memory_space=pl.ANY),
            pl.BlockSpec(memory_space=pl.ANY),
            pl.BlockSpec(memory_space=pltpu.VMEM),
        ],
        out_specs=pl.BlockSpec(memory_space=pltpu.VMEM),
        scratch_shapes=[
            pltpu.VMEM((N_DEV - 1, D_MODEL, BLK), _BF16),
            pltpu.VMEM((N_DEV - 1, BLK, D_MODEL), _BF16),
            pltpu.VMEM((2, G, G, 64, BLK), _F32),
            pltpu.VMEM((2, G, G, 64, BLK), _F32),
            pltpu.VMEM((SQ, BLK), _BF16),
            pltpu.VMEM((SQ, BLK), _BF16),
            pltpu.SemaphoreType.DMA((2, N_DEV)),
            pltpu.SemaphoreType.DMA((2, N_DEV)),
            pltpu.SemaphoreType.DMA((2, 2)),
        ],
        compiler_params=pltpu.CompilerParams(
            collective_id=0, vmem_limit_bytes=52 * 1024 * 1024),
    )(x2, wq, k2, v2, wo)
    return _perm(out).reshape(1, SQ, D_MODEL)


# device time: 123746 ns/iter; 1.0602x vs baseline; 1.0602x over previous
import jax
import jax.numpy as jnp
from jax import lax
from jax.experimental import pallas as pl
from jax.experimental.pallas import tpu as pltpu

N_DEV = 4
SQ = 1024
SKV = 1024
D_MODEL = 1024
HB = 8
DH = 128
BLK = HB * DH
G = 4
GS = SKV // G
SCALE = 0.08838834764831843

_F32 = jnp.float32
_BF16 = jnp.bfloat16

_MESH = pl.DeviceIdType.MESH


def _body(x_ref, wq_ref, k_hbm, v_hbm, wo_ref, out_ref,
          wq_buf, wo_buf, k_buf, v_buf, ctx_hop, ctx_keep,
          send_sems, recv_sems, kv_sems):
    my = lax.axis_index("i")
    left = lax.rem(my + (N_DEV - 1), N_DEV)
    right = lax.rem(my + 1, N_DEV)

    barrier = pltpu.get_barrier_semaphore()
    for nbr in (left, right):
        pl.semaphore_signal(barrier, inc=1, device_id=(nbr,),
                            device_id_type=_MESH)
    pl.semaphore_wait(barrier, 2)

    def wq_send(h, src):
        d = pltpu.make_async_remote_copy(
            src_ref=src, dst_ref=wq_buf.at[h],
            send_sem=send_sems.at[0, h], recv_sem=recv_sems.at[0, h],
            device_id=(right,), device_id_type=_MESH)
        d.start()
        return d

    def wo_send(h, src):
        d = pltpu.make_async_remote_copy(
            src_ref=src, dst_ref=wo_buf.at[h],
            send_sem=send_sems.at[1, h], recv_sem=recv_sems.at[1, h],
            device_id=(left,), device_id_type=_MESH)
        d.start()
        return d

    def kv_start(slot, blk):
        ds = []
        for r in range(G):
            ds.append(pltpu.make_async_copy(
                k_hbm.at[:, r, :, pl.ds(blk * BLK, BLK)],
                k_buf.at[slot, r], kv_sems.at[0, slot]))
            ds.append(pltpu.make_async_copy(
                v_hbm.at[:, r, :, pl.ds(blk * BLK, BLK)],
                v_buf.at[slot, r], kv_sems.at[1, slot]))
        for d in ds:
            d.start()
        return ds

    def kv_wait(ds):
        for d in ds:
            d.wait()

    def attn(wq_src, slot, ctx_dst, c0=0, nh=HB):
        q_full = lax.dot_general(x_ref[...], wq_src,
                                 (((1,), (0,)), ((), ())),
                                 preferred_element_type=_F32)
        for hd in range(nh):
            c = c0 + hd * DH
            q3 = (q_full[:, hd * DH:(hd + 1) * DH]
                  .astype(_BF16).reshape(G, GS, DH))
            k3 = k_buf[slot, :, :, :, c:c + DH].reshape(G, GS, DH)
            v3 = v_buf[slot, :, :, :, c:c + DH].reshape(G, GS, DH)
            s = lax.dot_general(q3, k3, (((2,), (2,)), ((0,), (0,))),
                                preferred_element_type=_F32)
            p = jnp.exp(s)
            p = (p * (1.0 / jnp.sum(p, axis=2, keepdims=True))).astype(_BF16)
            ctx = lax.dot_general(p, v3, (((2,), (1,)), ((0,), (0,))),
                                  preferred_element_type=_F32)
            ctx_dst[:, c:c + DH] = ctx.astype(_BF16).reshape(SQ, DH)

    def outproj(ctx_src, wo_src, first=False, col0=0):
        y = lax.dot_general(ctx_src[...], wo_src,
                            (((1,), (0,)), ((), ())),
                            preferred_element_type=_F32)
        w = y.shape[1]
        if first:
            out_ref[:, col0:col0 + w] = y
        else:
            out_ref[:, col0:col0 + w] += y

    snd = [wq_send(0, wq_ref), wo_send(0, wo_ref)]
    kv0 = kv_start(0, my)
    kv1 = kv_start(1, lax.rem(my + 3, N_DEV))

    kv_wait(kv0)
    attn(wq_ref[...], 0, ctx_hop)
    outproj(ctx_hop, wo_ref[...], first=True)

    snd[0].wait_recv()
    snd[1].wait_recv()
    snd.append(wq_send(1, wq_buf.at[0]))
    snd.append(wo_send(1, wo_buf.at[0]))
    kv2 = kv_start(0, lax.rem(my + 2, N_DEV))
    kv_wait(kv1)
    attn(wq_buf[0], 1, ctx_keep)

    snd[2].wait_recv()
    snd[3].wait_recv()
    halves = []
    for hf in range(2):
        cs = pl.ds(hf * 512, 512)
        d = pltpu.make_async_remote_copy(
            src_ref=wq_buf.at[1, :, cs], dst_ref=wq_buf.at[2, :, cs],
            send_sem=send_sems.at[0, 2 + hf],
            recv_sem=recv_sems.at[0, 2 + hf],
            device_id=(right,), device_id_type=_MESH)
        d.start()
        halves.append(d)
        snd.append(d)
        d = pltpu.make_async_remote_copy(
            src_ref=wo_buf.at[1, :, cs], dst_ref=wo_buf.at[2, :, cs],
            send_sem=send_sems.at[1, 2 + hf],
            recv_sem=recv_sems.at[1, 2 + hf],
            device_id=(left,), device_id_type=_MESH)
        d.start()
        halves.append(d)
        snd.append(d)
    kv3 = kv_start(1, lax.rem(my + 1, N_DEV))
    kv_wait(kv2)
    attn(wq_buf[1], 0, ctx_hop)
    outproj(ctx_hop, wo_buf[1])

    kv_wait(kv3)
    halves[0].wait_recv()
    attn(wq_buf[2, :, 0:512], 1, ctx_hop, 0, HB // 2)
    halves[1].wait_recv()
    outproj(ctx_keep, wo_buf[2, :, 0:512], col0=0)
    halves[2].wait_recv()
    attn(wq_buf[2, :, 512:1024], 1, ctx_hop, 512, HB // 2)
    halves[3].wait_recv()
    outproj(ctx_keep, wo_buf[2, :, 512:1024], col0=512)
    outproj(ctx_hop, wo_buf[0])

    for d in snd:
        d.wait_send()


def _perm(a):
    n = a.shape[0]
    return a.reshape(G, G, 64, -1).transpose(1, 0, 2, 3).reshape(n, -1)


def kernel(x, Wq, K_ext, V_ext, Wo):
    x2 = _perm(x.reshape(SQ, D_MODEL)).astype(_BF16)
    wq = (Wq * SCALE).astype(_BF16)
    wo = Wo.astype(_BF16)
    k2 = K_ext.reshape(SKV, N_DEV * BLK).astype(_BF16).reshape(
        G, G, 64, N_DEV * BLK)
    v2 = V_ext.reshape(SKV, N_DEV * BLK).astype(_BF16).reshape(
        G, G, 64, N_DEV * BLK)
    out = pl.pallas_call(
        _body,
        out_shape=jax.ShapeDtypeStruct((SQ, D_MODEL), _F32),
        in_specs=[
            pl.BlockSpec(memory_space=pltpu.VMEM),
            pl.BlockSpec(memory_space=pltpu.VMEM),
            pl.BlockSpec(memory_space=pl.ANY),
            pl.BlockSpec(memory_space=pl.ANY),
            pl.BlockSpec(memory_space=pltpu.VMEM),
        ],
        out_specs=pl.BlockSpec(memory_space=pltpu.VMEM),
        scratch_shapes=[
            pltpu.VMEM((N_DEV - 1, D_MODEL, BLK), _BF16),
            pltpu.VMEM((N_DEV - 1, BLK, D_MODEL), _BF16),
            pltpu.VMEM((2, G, G, 64, BLK), _BF16),
            pltpu.VMEM((2, G, G, 64, BLK), _BF16),
            pltpu.VMEM((SQ, BLK), _BF16),
            pltpu.VMEM((SQ, BLK), _BF16),
            pltpu.SemaphoreType.DMA((2, N_DEV)),
            pltpu.SemaphoreType.DMA((2, N_DEV)),
            pltpu.SemaphoreType.DMA((2, 2)),
        ],
        compiler_params=pltpu.CompilerParams(
            collective_id=0, vmem_limit_bytes=48 * 1024 * 1024),
    )(x2, wq, k2, v2, wo)
    return _perm(out).reshape(1, SQ, D_MODEL)
